# baseline (speedup 1.0000x reference)
import jax
import jax.numpy as jnp
import numpy as np
from jax import lax
from jax.experimental import pallas as pl
from jax.experimental.pallas import tpu as pltpu
from jax.experimental.pallas import tpu_sc as plsc

N = 10000
E = 160000
D = 256
DH = 128
TILES = 16
EPT = E // TILES
CH = 80
NCH = EPT // CH
ROWS_PT = N // TILES

f32 = jnp.float32
i32 = jnp.int32

_mesh = plsc.VectorSubcoreMesh(core_axis_name="c", subcore_axis_name="s")


K0BUF = 5
K0OUT = NCH // K0BUF
K2BUF = 2
K2OUT = (NCH - 1) // K2BUF


def _deg_body(ei_hbm, w_hbm, deg_hbm, dst_l, w_l, zb_v, deg_sh,
              d0, d1, d2, d3, d4, s0, s1, s2, s3, s4):
    c = lax.axis_index("c")
    s = lax.axis_index("s")
    didx = [d0, d1, d2, d3, d4]
    sems = [s0, s1, s2, s3, s4]

    @pl.when(c == 0)
    def _core0():
        pltpu.sync_copy(ei_hbm.at[pl.ds(E + s * EPT, EPT)], dst_l)
        pltpu.sync_copy(w_hbm.at[pl.ds(s * EPT, EPT)], w_l)

        def zb_body(i, carry):
            zb_v[pl.ds(i * 16, 16)] = jnp.zeros((16,), f32)
            return carry

        lax.fori_loop(0, 125, zb_body, 0)

        @pl.when(s < 5)
        def _zero():
            pltpu.sync_copy(zb_v, deg_sh.at[pl.ds(s * 2000, 2000)])

        plsc.subcore_barrier()

        def outer(g, carry):
            for b in range(K0BUF):
                cc = g * K0BUF + b

                @pl.when(g > 0)
                def _wait_prev(b=b):
                    pltpu.make_async_copy(
                        w_l.at[pl.ds(0, CH)], deg_sh.at[didx[b]], sems[b]
                    ).wait()

                for v in range(CH // 16):
                    didx[b][pl.ds(v * 16, 16)] = dst_l[pl.ds(cc * CH + v * 16, 16)]
                pltpu.async_copy(w_l.at[pl.ds(cc * CH, CH)],
                                 deg_sh.at[didx[b]], sems[b], add=True)
            return carry

        lax.fori_loop(0, K0OUT, outer, 0)
        for b in range(K0BUF):
            pltpu.make_async_copy(
                w_l.at[pl.ds(0, CH)], deg_sh.at[didx[b]], sems[b]
            ).wait()
        plsc.subcore_barrier()

        @pl.when(s == 0)
        def _writeback():
            pltpu.sync_copy(deg_sh, deg_hbm)


_deg_call = pl.kernel(
    _deg_body,
    mesh=_mesh,
    out_type=jax.ShapeDtypeStruct((N,), f32),
    scratch_types=[
        pltpu.VMEM((EPT,), i32),
        pltpu.VMEM((EPT,), f32),
        pltpu.VMEM((2000,), f32),
        pltpu.VMEM_SHARED((N,), f32),
    ] + [pltpu.VMEM((CH,), i32) for _ in range(K0BUF)]
      + [pltpu.SemaphoreType.DMA for _ in range(K0BUF)],
)


_RB = 400


def _mm_body(x_ref, w_ref, deg_ref, out_ref):
    dt = deg_ref[...] + 1.0
    dinv = jnp.where(dt > 0, lax.rsqrt(dt), 0.0)
    h = jnp.dot(x_ref[...], w_ref[...], preferred_element_type=f32) * dinv
    out_ref[0] = h[:, :DH].astype(jnp.bfloat16)
    out_ref[1] = h[:, DH:].astype(jnp.bfloat16)


_mm_call = pl.pallas_call(
    _mm_body,
    grid=(N // _RB,),
    in_specs=[
        pl.BlockSpec((_RB, D), lambda i: (i, 0)),
        pl.BlockSpec((D, D), lambda i: (0, 0)),
        pl.BlockSpec((_RB, 1), lambda i: (i, 0)),
    ],
    out_specs=pl.BlockSpec((2, _RB, DH), lambda i: (0, i, 0)),
    out_shape=jax.ShapeDtypeStruct((2, N, DH), jnp.bfloat16),
)


def _edge_body(hp_hbm, ei_hbm, w_hbm, zeros_hbm, acc_hbm, *scr):
    src_l, acc_sh = scr[0:2]
    sidx = list(scr[2:2 + K2BUF])
    didx = list(scr[2 + K2BUF:2 + 2 * K2BUF])
    wbuf = list(scr[2 + 2 * K2BUF:2 + 3 * K2BUF])
    rows = list(scr[2 + 3 * K2BUF:2 + 4 * K2BUF])
    scaled = list(scr[2 + 4 * K2BUF:2 + 5 * K2BUF])
    semg = list(scr[2 + 5 * K2BUF:2 + 6 * K2BUF])
    semw = list(scr[2 + 6 * K2BUF:2 + 7 * K2BUF])
    semd = list(scr[2 + 7 * K2BUF:2 + 8 * K2BUF])
    sems = list(scr[2 + 8 * K2BUF:2 + 9 * K2BUF])

    c = lax.axis_index("c")
    s = lax.axis_index("s")
    coff = c * N

    pltpu.sync_copy(ei_hbm.at[pl.ds(s * EPT, EPT)], src_l)

    @pl.when(s < 10)
    def _zero():
        pltpu.sync_copy(zeros_hbm, acc_sh.at[pl.ds(s * 1000, 1000)])

    plsc.subcore_barrier()

    def _process(cm1, prev):
        pltpu.make_async_copy(hp_hbm.at[sidx[prev]], rows[prev], semg[prev]).wait()
        pltpu.make_async_copy(w_hbm.at[pl.ds(0, CH)], wbuf[prev], semw[prev]).wait()

        def scale(jj, inner):
            wv = wbuf[prev][pl.ds(jj * 16, 16)]
            for l in range(16):
                j = jj * 16 + l
                wspl = jnp.full((16,), wv[l], f32)
                for v in range(4):
                    w32 = rows[prev][j, pl.ds(v * 16, 16)]
                    lo = lax.bitcast_convert_type(w32 << 16, f32)
                    hi = lax.bitcast_convert_type(w32 & jnp.int32(-65536), f32)
                    scaled[prev][j, pl.ds(v * 32, 16)] = lo * wspl
                    scaled[prev][j, pl.ds(v * 32 + 16, 16)] = hi * wspl
            return inner

        lax.fori_loop(0, CH // 16, scale, 0)
        pltpu.make_async_copy(ei_hbm.at[pl.ds(E, CH)], didx[prev], semd[prev]).wait()
        pltpu.async_copy(scaled[prev], acc_sh.at[didx[prev]], sems[prev], add=True)

    def _stage(b, cc):
        e0 = cc * CH
        pltpu.async_copy(w_hbm.at[pl.ds(s * EPT + e0, CH)], wbuf[b], semw[b])
        pltpu.async_copy(ei_hbm.at[pl.ds(E + s * EPT + e0, CH)], didx[b], semd[b])
        for v in range(CH // 16):
            sidx[b][pl.ds(v * 16, 16)] = src_l[pl.ds(e0 + v * 16, 16)] + coff
        pltpu.async_copy(hp_hbm.at[sidx[b]], rows[b], semg[b])

    def outer(g, carry):
        for b in range(K2BUF):
            cc = g * K2BUF + b

            @pl.when(g > 0)
            def _wait_prev(b=b):
                pltpu.make_async_copy(scaled[b], acc_sh.at[didx[b]], sems[b]).wait()

            _stage(b, cc)

            @pl.when(cc > 0)
            def _proc(cc=cc, prev=1 - b):
                _process(cc - 1, prev)
        return carry

    lax.fori_loop(0, K2OUT, outer, 0)
    pltpu.make_async_copy(scaled[0], acc_sh.at[didx[0]], sems[0]).wait()
    _stage(0, NCH - 1)
    _process(NCH - 2, 1)
    _process(NCH - 1, 0)
    for b in range(K2BUF):
        pltpu.make_async_copy(scaled[b], acc_sh.at[didx[b]], sems[b]).wait()
    plsc.subcore_barrier()

    @pl.when(s < 10)
    def _writeback():
        pltpu.sync_copy(acc_sh.at[pl.ds(s * 1000, 1000)],
                        acc_hbm.at[pl.ds(coff + s * 1000, 1000)])


_edge_call = pl.kernel(
    _edge_body,
    mesh=_mesh,
    compiler_params=pltpu.CompilerParams(use_tc_tiling_on_sc=False),
    out_type=jax.ShapeDtypeStruct((2 * N, DH), f32),
    scratch_types=[
        pltpu.VMEM((EPT,), i32),
        pltpu.VMEM_SHARED((N, DH), f32),
    ] + [pltpu.VMEM((CH,), i32) for _ in range(K2BUF)]
      + [pltpu.VMEM((CH,), i32) for _ in range(K2BUF)]
      + [pltpu.VMEM((CH,), f32) for _ in range(K2BUF)]
      + [pltpu.VMEM((CH, DH // 2), i32) for _ in range(K2BUF)]
      + [pltpu.VMEM((CH, DH), f32) for _ in range(K2BUF)]
      + [pltpu.SemaphoreType.DMA for _ in range(4 * K2BUF)],
)


def _bn_body(acc_ref, hp_ref, deg_ref, b_ref, gamma_ref, beta_ref, out_ref,
             l_sc, ps_sc, pq_sc):
    p = pl.program_id(0)
    i = pl.program_id(1)

    @pl.when(jnp.logical_and(p == 0, i == 0))
    def _init():
        ps_sc[...] = jnp.zeros((8, D), f32)
        pq_sc[...] = jnp.zeros((8, D), f32)

    @pl.when(p == 0)
    def _phase0():
        a = jnp.concatenate([acc_ref[0], acc_ref[1]], axis=1)
        h = jnp.concatenate([hp_ref[0], hp_ref[1]], axis=1).astype(f32)
        dt = deg_ref[...] + 1.0
        dinv = jnp.where(dt > 0, lax.rsqrt(dt), 0.0)
        pre = dinv * (a + h) + b_ref[...]
        lk = jnp.where(pre >= 0, pre, 0.1 * pre)
        l_sc[pl.ds(i * _RB, _RB), :] = lk
        ps_sc[...] += jnp.broadcast_to(jnp.sum(lk, axis=0, keepdims=True), (8, D))
        pq_sc[...] += jnp.broadcast_to(jnp.sum(lk * lk, axis=0, keepdims=True), (8, D))

    @pl.when(p == 1)
    def _phase1():
        lk = l_sc[pl.ds(i * _RB, _RB), :]
        mean = jnp.sum(ps_sc[...], axis=0, keepdims=True) / (8.0 * N)
        msq = jnp.sum(pq_sc[...], axis=0, keepdims=True) / (8.0 * N)
        var = msq - mean * mean
        inv = lax.rsqrt(var + 1e-5)
        out_ref[...] = (lk - mean) * (inv * gamma_ref[...]) + beta_ref[...]


_bn_call = pl.pallas_call(
    _bn_body,
    grid=(2, N // _RB),
    in_specs=[
        pl.BlockSpec((2, _RB, DH), lambda p, i: (0, i * (1 - p), 0)),
        pl.BlockSpec((2, _RB, DH), lambda p, i: (0, i * (1 - p), 0)),
        pl.BlockSpec((_RB, 1), lambda p, i: (i * (1 - p), 0)),
        pl.BlockSpec((1, D), lambda p, i: (0, 0)),
        pl.BlockSpec((1, D), lambda p, i: (0, 0)),
        pl.BlockSpec((1, D), lambda p, i: (0, 0)),
    ],
    out_specs=pl.BlockSpec((_RB, D), lambda p, i: (i, 0)),
    out_shape=jax.ShapeDtypeStruct((N, D), f32),
    scratch_shapes=[
        pltpu.VMEM((N, D), f32),
        pltpu.VMEM((8, D), f32),
        pltpu.VMEM((8, D), f32),
    ],
)


_PERM = np.empty((D,), np.int32)
for _g in range(D // 32):
    for _j in range(16):
        _PERM[32 * _g + 2 * _j] = 32 * _g + _j
        _PERM[32 * _g + 2 * _j + 1] = 32 * _g + 16 + _j
_INV = np.argsort(_PERM)


def kernel(x, edge_index, edge_attr, W, b, gamma, beta):
    ei = edge_index.reshape(2 * E)
    deg = _deg_call(ei, edge_attr)
    hp3 = _mm_call(x, W[:, _PERM], deg.reshape(N, 1))
    zeros = jnp.zeros((1000, DH), f32)
    hp_i32 = lax.bitcast_convert_type(
        hp3.reshape(2 * N, DH // 2, 2), jnp.int32)
    acc = _edge_call(hp_i32, ei, edge_attr, zeros)
    hp34 = hp3[:, :, _INV[:DH]]
    return _bn_call(acc.reshape(2, N, DH), hp34, deg.reshape(N, 1),
                    b.reshape(1, D), gamma.reshape(1, D), beta.reshape(1, D))


# speedup vs baseline: 2.1320x; 2.1320x over previous
import jax
import jax.numpy as jnp
from jax import lax
from jax.experimental import pallas as pl
from jax.experimental.pallas import tpu as pltpu
from jax.experimental.pallas import tpu_sc as plsc

N = 10000
E = 160000
D = 256
DH = 128
TILES = 16
EPT = E // TILES
CH = 80
NCH = EPT // CH
ROWS_PT = N // TILES

f32 = jnp.float32
i32 = jnp.int32

_mesh = plsc.VectorSubcoreMesh(core_axis_name="c", subcore_axis_name="s")


K0BUF = 5
K0OUT = NCH // K0BUF
K2BUF = 2
K2OUT = (NCH - 1) // K2BUF


def _deg_body(ei_hbm, w_hbm, deg_hbm, dst_l, w_l, zb_v, deg_sh,
              d0, d1, d2, d3, d4, s0, s1, s2, s3, s4):
    c = lax.axis_index("c")
    s = lax.axis_index("s")
    didx = [d0, d1, d2, d3, d4]
    sems = [s0, s1, s2, s3, s4]

    @pl.when(c == 0)
    def _core0():
        pltpu.sync_copy(ei_hbm.at[pl.ds(E + s * EPT, EPT)], dst_l)
        pltpu.sync_copy(w_hbm.at[pl.ds(s * EPT, EPT)], w_l)

        def zb_body(i, carry):
            zb_v[pl.ds(i * 16, 16)] = jnp.zeros((16,), f32)
            return carry

        lax.fori_loop(0, 125, zb_body, 0)

        @pl.when(s < 5)
        def _zero():
            pltpu.sync_copy(zb_v, deg_sh.at[pl.ds(s * 2000, 2000)])

        plsc.subcore_barrier()

        def outer(g, carry):
            for b in range(K0BUF):
                cc = g * K0BUF + b

                @pl.when(g > 0)
                def _wait_prev(b=b):
                    pltpu.make_async_copy(
                        w_l.at[pl.ds(0, CH)], deg_sh.at[didx[b]], sems[b]
                    ).wait()

                for v in range(CH // 16):
                    didx[b][pl.ds(v * 16, 16)] = dst_l[pl.ds(cc * CH + v * 16, 16)]
                pltpu.async_copy(w_l.at[pl.ds(cc * CH, CH)],
                                 deg_sh.at[didx[b]], sems[b], add=True)
            return carry

        lax.fori_loop(0, K0OUT, outer, 0)
        for b in range(K0BUF):
            pltpu.make_async_copy(
                w_l.at[pl.ds(0, CH)], deg_sh.at[didx[b]], sems[b]
            ).wait()
        plsc.subcore_barrier()

        @pl.when(s == 0)
        def _writeback():
            pltpu.sync_copy(deg_sh, deg_hbm)


_deg_call = pl.kernel(
    _deg_body,
    mesh=_mesh,
    out_type=jax.ShapeDtypeStruct((N,), f32),
    scratch_types=[
        pltpu.VMEM((EPT,), i32),
        pltpu.VMEM((EPT,), f32),
        pltpu.VMEM((2000,), f32),
        pltpu.VMEM_SHARED((N,), f32),
    ] + [pltpu.VMEM((CH,), i32) for _ in range(K0BUF)]
      + [pltpu.SemaphoreType.DMA for _ in range(K0BUF)],
)


_RB = 400


def _mm_body(x_ref, w_ref, deg_ref, out_ref):
    dt = deg_ref[...] + 1.0
    dinv = jnp.where(dt > 0, lax.rsqrt(dt), 0.0)
    h = jnp.dot(x_ref[...], w_ref[...], preferred_element_type=f32) * dinv
    out_ref[0] = h[:, :DH]
    out_ref[1] = h[:, DH:]


_mm_call = pl.pallas_call(
    _mm_body,
    grid=(N // _RB,),
    in_specs=[
        pl.BlockSpec((_RB, D), lambda i: (i, 0)),
        pl.BlockSpec((D, D), lambda i: (0, 0)),
        pl.BlockSpec((_RB, 1), lambda i: (i, 0)),
    ],
    out_specs=pl.BlockSpec((2, _RB, DH), lambda i: (0, i, 0)),
    out_shape=jax.ShapeDtypeStruct((2, N, DH), f32),
)


def _edge_body(hp_hbm, ei_hbm, w_hbm, zeros_hbm, acc_hbm, *scr):
    src_l, dst_l, acc_sh = scr[0:3]
    sidx = list(scr[3:3 + K2BUF])
    didx = list(scr[3 + K2BUF:3 + 2 * K2BUF])
    wbuf = list(scr[3 + 2 * K2BUF:3 + 3 * K2BUF])
    rows = list(scr[3 + 3 * K2BUF:3 + 4 * K2BUF])
    semg = list(scr[3 + 4 * K2BUF:3 + 5 * K2BUF])
    semw = list(scr[3 + 5 * K2BUF:3 + 6 * K2BUF])
    sems = list(scr[3 + 6 * K2BUF:3 + 7 * K2BUF])

    c = lax.axis_index("c")
    s = lax.axis_index("s")
    coff = c * N

    pltpu.sync_copy(ei_hbm.at[pl.ds(s * EPT, EPT)], src_l)
    pltpu.sync_copy(ei_hbm.at[pl.ds(E + s * EPT, EPT)], dst_l)

    @pl.when(s < 10)
    def _zero():
        pltpu.sync_copy(zeros_hbm, acc_sh.at[pl.ds(s * 1000, 1000)])

    plsc.subcore_barrier()

    def _process(cm1, prev):
        pltpu.make_async_copy(hp_hbm.at[sidx[prev]], rows[prev], semg[prev]).wait()
        pltpu.make_async_copy(w_hbm.at[pl.ds(0, CH)], wbuf[prev], semw[prev]).wait()

        def scale(jj, inner):
            wv = wbuf[prev][pl.ds(jj * 16, 16)]
            for l in range(16):
                j = jj * 16 + l
                wspl = jnp.full((16,), wv[l], f32)
                for v in range(8):
                    rows[prev][j, pl.ds(v * 16, 16)] = (
                        rows[prev][j, pl.ds(v * 16, 16)] * wspl)
            return inner

        lax.fori_loop(0, CH // 16, scale, 0)
        pltpu.async_copy(rows[prev], acc_sh.at[didx[prev]], sems[prev], add=True)

    def _stage(b, cc):
        e0 = cc * CH
        pltpu.async_copy(w_hbm.at[pl.ds(s * EPT + e0, CH)], wbuf[b], semw[b])
        for v in range(CH // 16):
            sidx[b][pl.ds(v * 16, 16)] = src_l[pl.ds(e0 + v * 16, 16)] + coff
            didx[b][pl.ds(v * 16, 16)] = dst_l[pl.ds(e0 + v * 16, 16)]
        pltpu.async_copy(hp_hbm.at[sidx[b]], rows[b], semg[b])

    def outer(g, carry):
        for b in range(K2BUF):
            cc = g * K2BUF + b

            @pl.when(g > 0)
            def _wait_prev(b=b):
                pltpu.make_async_copy(rows[b], acc_sh.at[didx[b]], sems[b]).wait()

            _stage(b, cc)

            @pl.when(cc > 0)
            def _proc(cc=cc, prev=1 - b):
                _process(cc - 1, prev)
        return carry

    lax.fori_loop(0, K2OUT, outer, 0)
    pltpu.make_async_copy(rows[0], acc_sh.at[didx[0]], sems[0]).wait()
    _stage(0, NCH - 1)
    _process(NCH - 2, 1)
    _process(NCH - 1, 0)
    for b in range(K2BUF):
        pltpu.make_async_copy(rows[b], acc_sh.at[didx[b]], sems[b]).wait()
    plsc.subcore_barrier()

    @pl.when(s < 10)
    def _writeback():
        pltpu.sync_copy(acc_sh.at[pl.ds(s * 1000, 1000)],
                        acc_hbm.at[pl.ds(coff + s * 1000, 1000)])


_edge_call = pl.kernel(
    _edge_body,
    mesh=_mesh,
    out_type=jax.ShapeDtypeStruct((2 * N, DH), f32),
    scratch_types=[
        pltpu.VMEM((EPT,), i32),
        pltpu.VMEM((EPT,), i32),
        pltpu.VMEM_SHARED((N, DH), f32),
    ] + [pltpu.VMEM((CH,), i32) for _ in range(K2BUF)]
      + [pltpu.VMEM((CH,), i32) for _ in range(K2BUF)]
      + [pltpu.VMEM((CH,), f32) for _ in range(K2BUF)]
      + [pltpu.VMEM((CH, DH), f32) for _ in range(K2BUF)]
      + [pltpu.SemaphoreType.DMA for _ in range(3 * K2BUF)],
)


def _bn_body(acc_ref, hp_ref, deg_ref, b_ref, gamma_ref, beta_ref, out_ref,
             l_sc, ps_sc, pq_sc):
    p = pl.program_id(0)
    i = pl.program_id(1)

    @pl.when(jnp.logical_and(p == 0, i == 0))
    def _init():
        ps_sc[...] = jnp.zeros((8, D), f32)
        pq_sc[...] = jnp.zeros((8, D), f32)

    @pl.when(p == 0)
    def _phase0():
        dt = deg_ref[...] + 1.0
        dinv = jnp.where(dt > 0, lax.rsqrt(dt), 0.0)
        ps = []
        pq = []
        for half in range(2):
            pre = dinv * (acc_ref[half] + hp_ref[half]) + b_ref[:, half * DH:(half + 1) * DH]
            lk = jnp.where(pre >= 0, pre, 0.1 * pre)
            l_sc[half, pl.ds(i * _RB, _RB), :] = lk
            ps.append(jnp.sum(lk, axis=0, keepdims=True))
            pq.append(jnp.sum(lk * lk, axis=0, keepdims=True))
        ps_sc[...] += jnp.broadcast_to(jnp.concatenate(ps, axis=1), (8, D))
        pq_sc[...] += jnp.broadcast_to(jnp.concatenate(pq, axis=1), (8, D))

    @pl.when(p == 1)
    def _phase1():
        mean = jnp.sum(ps_sc[...], axis=0, keepdims=True) / (8.0 * N)
        msq = jnp.sum(pq_sc[...], axis=0, keepdims=True) / (8.0 * N)
        var = msq - mean * mean
        inv = lax.rsqrt(var + 1e-5)
        scale = inv * gamma_ref[...]
        shift = beta_ref[...] - mean * scale
        for half in range(2):
            lk = l_sc[half, pl.ds(i * _RB, _RB), :]
            out_ref[:, pl.ds(half * DH, DH)] = (
                lk * scale[:, half * DH:(half + 1) * DH]
                + shift[:, half * DH:(half + 1) * DH])


_bn_call = pl.pallas_call(
    _bn_body,
    grid=(2, N // _RB),
    in_specs=[
        pl.BlockSpec((2, _RB, DH), lambda p, i: (0, i * (1 - p), 0)),
        pl.BlockSpec((2, _RB, DH), lambda p, i: (0, i * (1 - p), 0)),
        pl.BlockSpec((_RB, 1), lambda p, i: (i * (1 - p), 0)),
        pl.BlockSpec((1, D), lambda p, i: (0, 0)),
        pl.BlockSpec((1, D), lambda p, i: (0, 0)),
        pl.BlockSpec((1, D), lambda p, i: (0, 0)),
    ],
    out_specs=pl.BlockSpec((_RB, D), lambda p, i: (i, 0)),
    out_shape=jax.ShapeDtypeStruct((N, D), f32),
    scratch_shapes=[
        pltpu.VMEM((2, N, DH), f32),
        pltpu.VMEM((8, D), f32),
        pltpu.VMEM((8, D), f32),
    ],
)


def kernel(x, edge_index, edge_attr, W, b, gamma, beta):
    ei = edge_index.reshape(2 * E)
    deg = _deg_call(ei, edge_attr)
    hp3 = _mm_call(x, W, deg.reshape(N, 1))
    zeros = jnp.zeros((1000, DH), f32)
    acc = _edge_call(hp3.reshape(2 * N, DH), ei, edge_attr, zeros)
    return _bn_call(acc.reshape(2, N, DH), hp3, deg.reshape(N, 1),
                    b.reshape(1, D), gamma.reshape(1, D), beta.reshape(1, D))


# speedup vs baseline: 2.3407x; 1.0979x over previous
import jax
import jax.numpy as jnp
from jax import lax
from jax.experimental import pallas as pl
from jax.experimental.pallas import tpu as pltpu
from jax.experimental.pallas import tpu_sc as plsc

N = 10000
E = 160000
D = 256
DH = 128
TILES = 16
EPT = E // TILES
CH = 80
NCH = EPT // CH
ROWS_PT = N // TILES

f32 = jnp.float32
i32 = jnp.int32

_mesh = plsc.VectorSubcoreMesh(core_axis_name="c", subcore_axis_name="s")


K0BUF = 5
K0OUT = NCH // K0BUF
K2BUF = 3
K2OUT = 41


def _deg_body(ei_hbm, w_hbm, deg_hbm, dst_l, w_l, zb_v, deg_sh,
              d0, d1, d2, d3, d4, s0, s1, s2, s3, s4):
    c = lax.axis_index("c")
    s = lax.axis_index("s")
    didx = [d0, d1, d2, d3, d4]
    sems = [s0, s1, s2, s3, s4]

    @pl.when(c == 0)
    def _core0():
        pltpu.sync_copy(ei_hbm.at[pl.ds(E + s * EPT, EPT)], dst_l)
        pltpu.sync_copy(w_hbm.at[pl.ds(s * EPT, EPT)], w_l)

        def zb_body(i, carry):
            zb_v[pl.ds(i * 16, 16)] = jnp.zeros((16,), f32)
            return carry

        lax.fori_loop(0, 125, zb_body, 0)

        @pl.when(s < 5)
        def _zero():
            pltpu.sync_copy(zb_v, deg_sh.at[pl.ds(s * 2000, 2000)])

        plsc.subcore_barrier()

        def outer(g, carry):
            for b in range(K0BUF):
                cc = g * K0BUF + b

                @pl.when(g > 0)
                def _wait_prev(b=b):
                    pltpu.make_async_copy(
                        w_l.at[pl.ds(0, CH)], deg_sh.at[didx[b]], sems[b]
                    ).wait()

                for v in range(CH // 16):
                    didx[b][pl.ds(v * 16, 16)] = dst_l[pl.ds(cc * CH + v * 16, 16)]
                pltpu.async_copy(w_l.at[pl.ds(cc * CH, CH)],
                                 deg_sh.at[didx[b]], sems[b], add=True)
            return carry

        lax.fori_loop(0, K0OUT, outer, 0)
        for b in range(K0BUF):
            pltpu.make_async_copy(
                w_l.at[pl.ds(0, CH)], deg_sh.at[didx[b]], sems[b]
            ).wait()
        plsc.subcore_barrier()

        @pl.when(s == 0)
        def _writeback():
            pltpu.sync_copy(deg_sh, deg_hbm)


_deg_call = pl.kernel(
    _deg_body,
    mesh=_mesh,
    out_type=jax.ShapeDtypeStruct((N,), f32),
    scratch_types=[
        pltpu.VMEM((EPT,), i32),
        pltpu.VMEM((EPT,), f32),
        pltpu.VMEM((2000,), f32),
        pltpu.VMEM_SHARED((N,), f32),
    ] + [pltpu.VMEM((CH,), i32) for _ in range(K0BUF)]
      + [pltpu.SemaphoreType.DMA for _ in range(K0BUF)],
)


_RB = 400


def _mm_body(x_ref, w_ref, deg_ref, out_ref):
    dt = deg_ref[...] + 1.0
    dinv = jnp.where(dt > 0, lax.rsqrt(dt), 0.0)
    h = jnp.dot(x_ref[...], w_ref[...], preferred_element_type=f32) * dinv
    out_ref[0] = h[:, :DH]
    out_ref[1] = h[:, DH:]


_mm_call = pl.pallas_call(
    _mm_body,
    grid=(N // _RB,),
    in_specs=[
        pl.BlockSpec((_RB, D), lambda i: (i, 0)),
        pl.BlockSpec((D, D), lambda i: (0, 0)),
        pl.BlockSpec((_RB, 1), lambda i: (i, 0)),
    ],
    out_specs=pl.BlockSpec((2, _RB, DH), lambda i: (0, i, 0)),
    out_shape=jax.ShapeDtypeStruct((2, N, DH), f32),
)


def _edge_body(hp_hbm, ei_hbm, w_hbm, zeros_hbm, acc_hbm, *scr):
    src_l, acc_sh = scr[0:2]
    sidx = list(scr[2:2 + K2BUF])
    didx = list(scr[2 + K2BUF:2 + 2 * K2BUF])
    wbuf = list(scr[2 + 2 * K2BUF:2 + 3 * K2BUF])
    rows = list(scr[2 + 3 * K2BUF:2 + 4 * K2BUF])
    semg = list(scr[2 + 4 * K2BUF:2 + 5 * K2BUF])
    semw = list(scr[2 + 5 * K2BUF:2 + 6 * K2BUF])
    semd = list(scr[2 + 6 * K2BUF:2 + 7 * K2BUF])
    sems = list(scr[2 + 7 * K2BUF:2 + 8 * K2BUF])

    c = lax.axis_index("c")
    s = lax.axis_index("s")
    coff = c * N

    pltpu.sync_copy(ei_hbm.at[pl.ds(s * EPT, EPT)], src_l)

    @pl.when(s < 10)
    def _zero():
        pltpu.sync_copy(zeros_hbm, acc_sh.at[pl.ds(s * 1000, 1000)])

    plsc.subcore_barrier()

    def _process(cm1, prev):
        pltpu.make_async_copy(hp_hbm.at[sidx[prev]], rows[prev], semg[prev]).wait()
        pltpu.make_async_copy(w_hbm.at[pl.ds(0, CH)], wbuf[prev], semw[prev]).wait()

        def scale(jj, inner):
            wv = wbuf[prev][pl.ds(jj * 16, 16)]
            for l in range(16):
                j = jj * 16 + l
                wspl = jnp.full((16,), wv[l], f32)
                for v in range(8):
                    rows[prev][j, pl.ds(v * 16, 16)] = (
                        rows[prev][j, pl.ds(v * 16, 16)] * wspl)
            return inner

        lax.fori_loop(0, CH // 16, scale, 0)
        pltpu.make_async_copy(ei_hbm.at[pl.ds(E, CH)], didx[prev], semd[prev]).wait()
        pltpu.async_copy(rows[prev], acc_sh.at[didx[prev]], sems[prev], add=True)

    def _stage(b, cc):
        e0 = cc * CH
        pltpu.async_copy(w_hbm.at[pl.ds(s * EPT + e0, CH)], wbuf[b], semw[b])
        pltpu.async_copy(ei_hbm.at[pl.ds(E + s * EPT + e0, CH)], didx[b], semd[b])
        for v in range(CH // 16):
            sidx[b][pl.ds(v * 16, 16)] = src_l[pl.ds(e0 + v * 16, 16)] + coff
        pltpu.async_copy(hp_hbm.at[sidx[b]], rows[b], semg[b])

    def outer(g, carry):
        for b in range(K2BUF):
            cc = g * K2BUF + b

            @pl.when(g > 0)
            def _wait_prev(b=b):
                pltpu.make_async_copy(rows[b], acc_sh.at[didx[b]], sems[b]).wait()

            _stage(b, cc)

            @pl.when(cc > 0)
            def _proc(cc=cc, prev=(b - 1) % K2BUF):
                _process(cc - 1, prev)
        return carry

    lax.fori_loop(0, K2OUT, outer, 0)
    pltpu.make_async_copy(rows[0], acc_sh.at[didx[0]], sems[0]).wait()
    _stage(0, NCH - 2)
    _process(NCH - 3, 2)
    pltpu.make_async_copy(rows[1], acc_sh.at[didx[1]], sems[1]).wait()
    _stage(1, NCH - 1)
    _process(NCH - 2, 0)
    _process(NCH - 1, 1)
    for b in range(K2BUF):
        pltpu.make_async_copy(rows[b], acc_sh.at[didx[b]], sems[b]).wait()
    plsc.subcore_barrier()

    @pl.when(s < 10)
    def _writeback():
        pltpu.sync_copy(acc_sh.at[pl.ds(s * 1000, 1000)],
                        acc_hbm.at[pl.ds(coff + s * 1000, 1000)])


_edge_call = pl.kernel(
    _edge_body,
    mesh=_mesh,
    out_type=jax.ShapeDtypeStruct((2 * N, DH), f32),
    scratch_types=[
        pltpu.VMEM((EPT,), i32),
        pltpu.VMEM_SHARED((N, DH), f32),
    ] + [pltpu.VMEM((CH,), i32) for _ in range(K2BUF)]
      + [pltpu.VMEM((CH,), i32) for _ in range(K2BUF)]
      + [pltpu.VMEM((CH,), f32) for _ in range(K2BUF)]
      + [pltpu.VMEM((CH, DH), f32) for _ in range(K2BUF)]
      + [pltpu.SemaphoreType.DMA for _ in range(4 * K2BUF)],
)


def _bn_body(acc_ref, hp_ref, deg_ref, b_ref, gamma_ref, beta_ref, out_ref,
             l_sc, ps_sc, pq_sc):
    p = pl.program_id(0)
    i = pl.program_id(1)

    @pl.when(jnp.logical_and(p == 0, i == 0))
    def _init():
        ps_sc[...] = jnp.zeros((8, D), f32)
        pq_sc[...] = jnp.zeros((8, D), f32)

    @pl.when(p == 0)
    def _phase0():
        dt = deg_ref[...] + 1.0
        dinv = jnp.where(dt > 0, lax.rsqrt(dt), 0.0)
        ps = []
        pq = []
        for half in range(2):
            pre = dinv * (acc_ref[half] + hp_ref[half]) + b_ref[:, half * DH:(half + 1) * DH]
            lk = jnp.where(pre >= 0, pre, 0.1 * pre)
            l_sc[half, pl.ds(i * _RB, _RB), :] = lk
            ps.append(jnp.sum(lk, axis=0, keepdims=True))
            pq.append(jnp.sum(lk * lk, axis=0, keepdims=True))
        ps_sc[...] += jnp.broadcast_to(jnp.concatenate(ps, axis=1), (8, D))
        pq_sc[...] += jnp.broadcast_to(jnp.concatenate(pq, axis=1), (8, D))

    @pl.when(p == 1)
    def _phase1():
        mean = jnp.sum(ps_sc[...], axis=0, keepdims=True) / (8.0 * N)
        msq = jnp.sum(pq_sc[...], axis=0, keepdims=True) / (8.0 * N)
        var = msq - mean * mean
        inv = lax.rsqrt(var + 1e-5)
        scale = inv * gamma_ref[...]
        shift = beta_ref[...] - mean * scale
        for half in range(2):
            lk = l_sc[half, pl.ds(i * _RB, _RB), :]
            out_ref[:, pl.ds(half * DH, DH)] = (
                lk * scale[:, half * DH:(half + 1) * DH]
                + shift[:, half * DH:(half + 1) * DH])


_bn_call = pl.pallas_call(
    _bn_body,
    grid=(2, N // _RB),
    in_specs=[
        pl.BlockSpec((2, _RB, DH), lambda p, i: (0, i * (1 - p), 0)),
        pl.BlockSpec((2, _RB, DH), lambda p, i: (0, i * (1 - p), 0)),
        pl.BlockSpec((_RB, 1), lambda p, i: (i * (1 - p), 0)),
        pl.BlockSpec((1, D), lambda p, i: (0, 0)),
        pl.BlockSpec((1, D), lambda p, i: (0, 0)),
        pl.BlockSpec((1, D), lambda p, i: (0, 0)),
    ],
    out_specs=pl.BlockSpec((_RB, D), lambda p, i: (i, 0)),
    out_shape=jax.ShapeDtypeStruct((N, D), f32),
    scratch_shapes=[
        pltpu.VMEM((2, N, DH), f32),
        pltpu.VMEM((8, D), f32),
        pltpu.VMEM((8, D), f32),
    ],
)


def kernel(x, edge_index, edge_attr, W, b, gamma, beta):
    ei = edge_index.reshape(2 * E)
    deg = _deg_call(ei, edge_attr)
    hp3 = _mm_call(x, W, deg.reshape(N, 1))
    zeros = jnp.zeros((1000, DH), f32)
    acc = _edge_call(hp3.reshape(2 * N, DH), ei, edge_attr, zeros)
    return _bn_call(acc.reshape(2, N, DH), hp3, deg.reshape(N, 1),
                    b.reshape(1, D), gamma.reshape(1, D), beta.reshape(1, D))
